# baseline (speedup 1.0000x reference)
import functools

import jax
import jax.numpy as jnp
from jax import lax
from jax.experimental import pallas as pl
from jax.experimental.pallas import tpu as pltpu
from jax.experimental.pallas import tpu_sc as plsc

N = 8192
K = 32
R = 32
NBLK = N // R
F = 16
E = N * K

def _round_bf16(v):
    i = lax.bitcast_convert_type(v, jnp.int32)
    r = i + jnp.int32(0x7FFF) + ((i >> 16) & 1)
    r = lax.bitwise_and(r, jnp.int32(-65536))
    return lax.bitcast_convert_type(r, jnp.float32)


def _topk_body(loc_ref, posT_ref, nbr_ref, feat_ref, d2_ref):
    b = pl.program_id(0)
    li = loc_ref[...]
    w = li[:, 4:5]
    h = li[:, 5:6]
    cx = li[:, 6:7]
    cy = li[:, 7:8]
    posT = posT_ref[...]
    xc = posT[0:1, :]
    yc = posT[1:2, :]
    sq_c = xc * xc + yc * yc
    sq_r = cx * cx + cy * cy
    dot = (_round_bf16(cx) * _round_bf16(xc)
           + _round_bf16(cy) * _round_bf16(yc))
    d2 = (sq_r + sq_c) - 2.0 * dot
    colI = lax.broadcasted_iota(jnp.int32, (R, N), 1)
    rowI = lax.broadcasted_iota(jnp.int32, (R, N), 0) + b * R
    inf = jnp.float32(jnp.inf)
    d2_ref[...] = jnp.where(colI == rowI, inf, d2)
    big = jnp.int32(2**30)
    lane = lax.broadcasted_iota(jnp.int32, (R, K), 1)

    lane128 = lax.broadcasted_iota(jnp.int32, (R, 128), 1)
    nch = N // 128
    Q = 8
    NR = K // Q

    def round_fn(r, carry):
        vlast, clast, acc = carry
        ms = [jnp.full((R, 128), inf) for _ in range(Q)]
        cs = [jnp.full((R, 128), big) for _ in range(Q)]
        for c in range(nch):
            x = d2_ref[:, c * 128:(c + 1) * 128]
            colx = lane128 + jnp.int32(c * 128)
            pred = (x > vlast) | ((x == vlast) & (colx > clast))
            x = jnp.where(pred, x, inf)
            cx = colx
            for i in range(Q):
                lt = x < ms[i]
                y = jnp.maximum(ms[i], x)
                cy = jnp.where(lt, cs[i], cx)
                ms[i] = jnp.minimum(ms[i], x)
                cs[i] = jnp.where(lt, cx, cs[i])
                x, cx = y, cy
        p = jnp.zeros((R, 128), jnp.int32)
        for j in range(Q):
            hv, hc = ms[Q - 1], cs[Q - 1]
            for i in range(Q - 2, -1, -1):
                sel = p == i
                hv = jnp.where(sel, ms[i], hv)
                hc = jnp.where(sel, cs[i], hc)
            gm = jnp.min(hv, axis=1, keepdims=True)
            a = jnp.min(jnp.where(hv == gm, hc, big), axis=1,
                        keepdims=True)
            p = p + (hc == a).astype(jnp.int32)
            acc = jnp.where(lane == r * Q + j, a, acc)
            vlast, clast = gm, a
        return vlast, clast, acc

    _, _, acc = lax.fori_loop(
        0, NR, round_fn,
        (jnp.full((R, 1), -inf), jnp.full((R, 1), jnp.int32(-1)),
         jnp.zeros((R, K), jnp.int32)))
    nbr_ref[...] = acc
    feat_ref[...] = jnp.concatenate(
        [cx, cy, w, h, jnp.log(w), jnp.log(h),
         jnp.zeros((R, F - 6), jnp.float32)], axis=1)


def _run_topk(location_info, posT):
    return pl.pallas_call(
        _topk_body,
        grid=(NBLK,),
        in_specs=[
            pl.BlockSpec((R, 10), lambda b: (b, 0)),
            pl.BlockSpec((2, N), lambda b: (0, 0)),
        ],
        out_specs=[
            pl.BlockSpec((R, K), lambda b: (b, 0)),
            pl.BlockSpec((R, F), lambda b: (b, 0)),
        ],
        out_shape=[
            jax.ShapeDtypeStruct((N, K), jnp.int32),
            jax.ShapeDtypeStruct((N, F), jnp.float32),
        ],
        scratch_shapes=[pltpu.VMEM((R, N), jnp.float32)],
    )(location_info, posT)


def _run_gather(feat, idx2d):
    info = plsc.get_sparse_core_info()
    nw = info.num_cores * info.num_subcores
    rows_per_w = E // nw
    grp = 8
    ngrp = rows_per_w // (grp * 128)
    mesh = plsc.VectorSubcoreMesh(core_axis_name="c", subcore_axis_name="s")

    @functools.partial(
        pl.kernel, mesh=mesh,
        compiler_params=pltpu.CompilerParams(use_tc_tiling_on_sc=False),
        out_type=jax.ShapeDtypeStruct((E, F), jnp.float32),
        scratch_types=[
            pltpu.VMEM((rows_per_w // 128, 128), jnp.int32),
            pltpu.VMEM((grp * 128, F), jnp.float32),
            pltpu.SemaphoreType.DMA,
        ],
    )
    def k(feat_hbm, idx_hbm, out_hbm, idx_v, rows_v, sem):
        wid = lax.axis_index("s") * info.num_cores + lax.axis_index("c")
        nchunk = rows_per_w // 128
        pltpu.sync_copy(idx_hbm.at[pl.ds(wid * nchunk, nchunk)], idx_v)

        def group(g, carry):
            descs = []
            for j in range(grp):
                c = g * grp + j
                descs.append(pltpu.async_copy(
                    feat_hbm.at[idx_v.at[c]],
                    rows_v.at[pl.ds(j * 128, 128)], sem))
            for d in descs:
                d.wait()
            pltpu.sync_copy(
                rows_v,
                out_hbm.at[pl.ds(wid * rows_per_w + g * grp * 128, grp * 128)])
            return carry

        lax.fori_loop(0, ngrp, group, 0)

    return k(feat, idx2d)


EB = 2048
TB = EB // K


def _mlp_body(s_ref, t_ref, w1_ref, b1_ref, w2_ref, b2_ref, out_ref):
    s = s_ref[...].reshape(TB, K, F)
    t = t_ref[...][:, None, :]
    f1 = 2.0 * (s[:, :, 0:1] - t[:, :, 0:1]) / (s[:, :, 2:3] + t[:, :, 2:3])
    f2 = 2.0 * (s[:, :, 1:2] - t[:, :, 1:2]) / (s[:, :, 3:4] + t[:, :, 3:4])
    f3 = s[:, :, 4:5] - t[:, :, 4:5]
    f4 = s[:, :, 5:6] - t[:, :, 5:6]
    w1 = w1_ref[...]
    h = (f1 * w1[0:1, :][None] + f2 * w1[1:2, :][None]
         + f3 * w1[2:3, :][None] + f4 * w1[3:4, :][None]
         + b1_ref[...][None])
    h = jnp.maximum(h, 0.0).reshape(EB, 16)
    out_ref[...] = jnp.dot(h, w2_ref[...],
                           preferred_element_type=jnp.float32) + b2_ref[...]


def _run_mlp(s_rows, feat, W1, b1, W2, b2):
    return pl.pallas_call(
        _mlp_body,
        grid=(E // EB,),
        in_specs=[
            pl.BlockSpec((EB, F), lambda e: (e, 0)),
            pl.BlockSpec((TB, F), lambda e: (e, 0)),
            pl.BlockSpec((4, 16), lambda e: (0, 0)),
            pl.BlockSpec((1, 16), lambda e: (0, 0)),
            pl.BlockSpec((16, 32), lambda e: (0, 0)),
            pl.BlockSpec((1, 32), lambda e: (0, 0)),
        ],
        out_specs=pl.BlockSpec((EB, 32), lambda e: (e, 0)),
        out_shape=jax.ShapeDtypeStruct((E, 32), jnp.float32),
    )(s_rows, feat, W1, b1, W2, b2)


def kernel(x, location_info, W1, b1, W2, b2, k):
    del x, k
    posT = location_info[:, 6:8].T
    nbr, feat = _run_topk(location_info, posT)
    src = nbr.reshape(-1)
    s_rows = _run_gather(feat, src.reshape(E // 128, 128))
    edge_attr = _run_mlp(s_rows, feat, W1, b1.reshape(1, 16),
                         W2, b2.reshape(1, 32))
    tgt = jnp.repeat(jnp.arange(N, dtype=jnp.int32), K)
    edge_index = jnp.stack([src, tgt], axis=0)
    return edge_index, edge_attr


# speedup vs baseline: 1.2355x; 1.2355x over previous
import functools

import jax
import jax.numpy as jnp
from jax import lax
from jax.experimental import pallas as pl
from jax.experimental.pallas import tpu as pltpu
from jax.experimental.pallas import tpu_sc as plsc

N = 8192
K = 32
R = 64
NBLK = N // R
F = 16
E = N * K

def _round_bf16(v):
    i = lax.bitcast_convert_type(v, jnp.int32)
    r = i + jnp.int32(0x7FFF) + ((i >> 16) & 1)
    r = lax.bitwise_and(r, jnp.int32(-65536))
    return lax.bitcast_convert_type(r, jnp.float32)


def _topk_body(loc_ref, posT_ref, nbr_ref, feat_ref, d2_ref):
    b = pl.program_id(0)
    li = loc_ref[...]
    w = li[:, 4:5]
    h = li[:, 5:6]
    cx = li[:, 6:7]
    cy = li[:, 7:8]
    posT = posT_ref[...]
    sq_r = cx * cx + cy * cy
    cxb = _round_bf16(cx)
    cyb = _round_bf16(cy)
    rowv = lax.broadcasted_iota(jnp.int32, (R, 1), 0) + b * R
    inf = jnp.float32(jnp.inf)
    big = jnp.int32(2**30)
    lane = lax.broadcasted_iota(jnp.int32, (R, K), 1)
    lane128 = lax.broadcasted_iota(jnp.int32, (R, 128), 1)
    nch = N // 128
    Q = 8
    NR = K // Q

    def insert(ms, cs, x, ix):
        for i in range(Q):
            lt = x < ms[i]
            y = jnp.maximum(ms[i], x)
            iy = jnp.where(lt, cs[i], ix)
            ms[i] = jnp.minimum(ms[i], x)
            cs[i] = jnp.where(lt, ix, cs[i])
            x, ix = y, iy

    def pops(ms, cs, vlast, clast, acc, t0):
        p = jnp.zeros((R, 128), jnp.int32)
        for j in range(Q):
            hv, hc = ms[Q - 1], cs[Q - 1]
            for i in range(Q - 2, -1, -1):
                sel = p == i
                hv = jnp.where(sel, ms[i], hv)
                hc = jnp.where(sel, cs[i], hc)
            gm = jnp.min(hv, axis=1, keepdims=True)
            a = jnp.min(jnp.where(hv == gm, hc, big), axis=1,
                        keepdims=True)
            p = p + (hc == a).astype(jnp.int32)
            acc = jnp.where(lane == t0 + j, a, acc)
            vlast, clast = gm, a
        return vlast, clast, acc

    xc = posT[0:1, :]
    yc = posT[1:2, :]
    sq_c = xc * xc + yc * yc
    dot = (cxb * _round_bf16(xc)
           + cyb * _round_bf16(yc))
    d2 = (sq_r + sq_c) - 2.0 * dot
    colI = lax.broadcasted_iota(jnp.int32, (R, N), 1)
    d2_ref[...] = jnp.where(colI == rowv, inf, d2)

    def round_fn(r, carry):
        vlast, clast, acc = carry
        ms = [jnp.full((R, 128), inf) for _ in range(Q)]
        cs = [jnp.full((R, 128), big) for _ in range(Q)]
        for c in range(nch):
            x = d2_ref[:, c * 128:(c + 1) * 128]
            colx = lane128 + jnp.int32(c * 128)
            pred = (x > vlast) | ((x == vlast) & (colx > clast))
            x = jnp.where(pred, x, inf)
            insert(ms, cs, x, colx)
        return pops(ms, cs, vlast, clast, acc, r * Q)

    _, _, acc = lax.fori_loop(
        0, NR, round_fn,
        (jnp.full((R, 1), -inf), jnp.full((R, 1), jnp.int32(-1)),
         jnp.zeros((R, K), jnp.int32)))
    nbr_ref[...] = acc
    feat_ref[...] = jnp.concatenate(
        [cx, cy, w, h, jnp.log(w), jnp.log(h),
         jnp.zeros((R, F - 6), jnp.float32)], axis=1)


def _run_topk(location_info, posT):
    return pl.pallas_call(
        _topk_body,
        grid=(NBLK,),
        in_specs=[
            pl.BlockSpec((R, 10), lambda b: (b, 0)),
            pl.BlockSpec((2, N), lambda b: (0, 0)),
        ],
        out_specs=[
            pl.BlockSpec((R, K), lambda b: (b, 0)),
            pl.BlockSpec((R, F), lambda b: (b, 0)),
        ],
        out_shape=[
            jax.ShapeDtypeStruct((N, K), jnp.int32),
            jax.ShapeDtypeStruct((N, F), jnp.float32),
        ],
        scratch_shapes=[pltpu.VMEM((R, N), jnp.float32)],
    )(location_info, posT)


def _run_gather(feat, idx2d):
    info = plsc.get_sparse_core_info()
    nw = info.num_cores * info.num_subcores
    rows_per_w = E // nw
    grp = 8
    ngrp = rows_per_w // (grp * 128)
    mesh = plsc.VectorSubcoreMesh(core_axis_name="c", subcore_axis_name="s")

    @functools.partial(
        pl.kernel, mesh=mesh,
        compiler_params=pltpu.CompilerParams(use_tc_tiling_on_sc=False),
        out_type=jax.ShapeDtypeStruct((E, F), jnp.float32),
        scratch_types=[
            pltpu.VMEM((rows_per_w // 128, 128), jnp.int32),
            pltpu.VMEM((grp * 128, F), jnp.float32),
            pltpu.SemaphoreType.DMA,
        ],
    )
    def k(feat_hbm, idx_hbm, out_hbm, idx_v, rows_v, sem):
        wid = lax.axis_index("s") * info.num_cores + lax.axis_index("c")
        nchunk = rows_per_w // 128
        pltpu.sync_copy(idx_hbm.at[pl.ds(wid * nchunk, nchunk)], idx_v)

        def group(g, carry):
            descs = []
            for j in range(grp):
                c = g * grp + j
                descs.append(pltpu.async_copy(
                    feat_hbm.at[idx_v.at[c]],
                    rows_v.at[pl.ds(j * 128, 128)], sem))
            for d in descs:
                d.wait()
            pltpu.sync_copy(
                rows_v,
                out_hbm.at[pl.ds(wid * rows_per_w + g * grp * 128, grp * 128)])
            return carry

        lax.fori_loop(0, ngrp, group, 0)

    return k(feat, idx2d)


EB = 2048
TB = EB // K


def _mlp_body(s_ref, t_ref, w1_ref, b1_ref, w2_ref, b2_ref, out_ref):
    s = s_ref[...].reshape(TB, K, F)
    t = t_ref[...][:, None, :]
    f1 = 2.0 * (s[:, :, 0:1] - t[:, :, 0:1]) / (s[:, :, 2:3] + t[:, :, 2:3])
    f2 = 2.0 * (s[:, :, 1:2] - t[:, :, 1:2]) / (s[:, :, 3:4] + t[:, :, 3:4])
    f3 = s[:, :, 4:5] - t[:, :, 4:5]
    f4 = s[:, :, 5:6] - t[:, :, 5:6]
    w1 = w1_ref[...]
    h = (f1 * w1[0:1, :][None] + f2 * w1[1:2, :][None]
         + f3 * w1[2:3, :][None] + f4 * w1[3:4, :][None]
         + b1_ref[...][None])
    h = jnp.maximum(h, 0.0).reshape(EB, 16)
    out_ref[...] = jnp.dot(h, w2_ref[...],
                           preferred_element_type=jnp.float32) + b2_ref[...]


def _run_mlp(s_rows, feat, W1, b1, W2, b2):
    return pl.pallas_call(
        _mlp_body,
        grid=(E // EB,),
        in_specs=[
            pl.BlockSpec((EB, F), lambda e: (e, 0)),
            pl.BlockSpec((TB, F), lambda e: (e, 0)),
            pl.BlockSpec((4, 16), lambda e: (0, 0)),
            pl.BlockSpec((1, 16), lambda e: (0, 0)),
            pl.BlockSpec((16, 32), lambda e: (0, 0)),
            pl.BlockSpec((1, 32), lambda e: (0, 0)),
        ],
        out_specs=pl.BlockSpec((EB, 32), lambda e: (e, 0)),
        out_shape=jax.ShapeDtypeStruct((E, 32), jnp.float32),
    )(s_rows, feat, W1, b1, W2, b2)


def kernel(x, location_info, W1, b1, W2, b2, k):
    del x, k
    posT = location_info[:, 6:8].T
    nbr, feat = _run_topk(location_info, posT)
    src = nbr.reshape(-1)
    s_rows = _run_gather(feat, src.reshape(E // 128, 128))
    edge_attr = _run_mlp(s_rows, feat, W1, b1.reshape(1, 16),
                         W2, b2.reshape(1, 32))
    tgt = jnp.repeat(jnp.arange(N, dtype=jnp.int32), K)
    edge_index = jnp.stack([src, tgt], axis=0)
    return edge_index, edge_attr
